# Initial kernel scaffold; baseline (speedup 1.0000x reference)
import jax
import jax.numpy as jnp
from jax.experimental import pallas as pl


def kernel(task_embedding, skill_matrix, top_k):
    raise NotImplementedError("write your pallas kernel here")


# speedup vs baseline: 1.2604x; 1.2604x over previous
import functools

import jax
import jax.numpy as jnp
from jax import lax
from jax.experimental import pallas as pl
from jax.experimental.pallas import tpu as pltpu
from jax.experimental.pallas import tpu_sc as plsc

D = 64
K = 8
EPS = 1e-8
NEG_INF = float("-inf")
INT_MAX = 2**31 - 1


def _sims_body(te_ref, x_ref, o_ref):
    x = x_ref[...]
    te = te_ref[...]
    xb = x.astype(jnp.bfloat16).astype(jnp.float32)
    d = jnp.sum(xb * te[None, :], axis=1)
    n2 = jnp.sum(x * x, axis=1)
    o_ref[...] = d / jnp.maximum(jnp.sqrt(n2), EPS)


def _sims_tc(skill_matrix, te_scaled):
    n = skill_matrix.shape[0]
    blk = 8192
    grid = pl.cdiv(n, blk)
    return pl.pallas_call(
        _sims_body,
        grid=(grid,),
        in_specs=[
            pl.BlockSpec((D,), lambda i: (0,)),
            pl.BlockSpec((blk, D), lambda i: (i, 0)),
        ],
        out_specs=pl.BlockSpec((blk,), lambda i: (i,)),
        out_shape=jax.ShapeDtypeStruct((n,), jnp.float32),
    )(te_scaled, skill_matrix)


def _tile_topk_body(nw, base, sims_hbm, cv_hbm, ci_hbm, chunk, ov, oi):
    info = plsc.get_sparse_core_info()
    nc = info.num_cores
    wid = lax.axis_index("s") * nc + lax.axis_index("c")
    n = sims_hbm.shape[0]
    tail = n - (nw - 1) * base
    off = wid * base

    @pl.when(wid < nw - 1)
    def _():
        pltpu.sync_copy(sims_hbm.at[pl.ds(off, base)], chunk.at[pl.ds(0, base)])

    @pl.when(wid == nw - 1)
    def _():
        pltpu.sync_copy(sims_hbm.at[pl.ds(off, tail)], chunk.at[pl.ds(0, tail)])

    nv = jnp.where(wid == nw - 1, tail // 16, base // 16)
    lane = lax.iota(jnp.int32, 16)
    res_v = jnp.full((16,), NEG_INF, jnp.float32)
    res_i = jnp.zeros((16,), jnp.int32)

    for r in range(K):
        def body(c, carry):
            acc_v, acc_i = carry
            v = chunk[pl.ds(c * 16, 16)]
            m = v > acc_v
            return jnp.where(m, v, acc_v), jnp.where(m, c, acc_i)

        acc_v, acc_i = lax.fori_loop(
            0, nv,
            body,
            (jnp.full((16,), NEG_INF, jnp.float32), jnp.zeros((16,), jnp.int32)),
        )
        tmax = jnp.max(acc_v)
        gcand = jnp.where(acc_v == tmax, acc_i * 16 + lane, INT_MAX)
        g = jnp.min(gcand)
        res_v = jnp.where(lane == r, tmax, res_v)
        res_i = jnp.where(lane == r, g + off, res_i)
        plsc.store_scatter(chunk, [jnp.full((16,), g, jnp.int32)],
                           jnp.full((16,), NEG_INF, jnp.float32),
                           mask=lane == 0)

    ov[...] = res_v
    oi[...] = res_i
    pltpu.sync_copy(ov, cv_hbm.at[wid])
    pltpu.sync_copy(oi, ci_hbm.at[wid])


def _tile_topk(sims):
    n = sims.shape[0]
    info = plsc.get_sparse_core_info()
    nw = info.num_cores * info.num_subcores
    base = (n // (nw * 16)) * 16
    tail = n - (nw - 1) * base
    mesh = plsc.VectorSubcoreMesh(core_axis_name="c", subcore_axis_name="s")
    kern = pl.kernel(
        functools.partial(_tile_topk_body, nw, base),
        mesh=mesh,
        compiler_params=pltpu.CompilerParams(needs_layout_passes=False),
        out_type=(
            jax.ShapeDtypeStruct((nw, 16), jnp.float32),
            jax.ShapeDtypeStruct((nw, 16), jnp.int32),
        ),
        scratch_types=[
            pltpu.VMEM((tail,), jnp.float32),
            pltpu.VMEM((16,), jnp.float32),
            pltpu.VMEM((16,), jnp.int32),
        ],
    )
    return kern(sims)


def _merge_body(ncand, cv_hbm, ci_hbm, ov_hbm, oi_hbm, mv, mi, ov, oi):
    info = plsc.get_sparse_core_info()
    nc = info.num_cores
    wid = lax.axis_index("s") * nc + lax.axis_index("c")

    @pl.when(wid == 0)
    def _():
        pltpu.sync_copy(cv_hbm, mv)
        pltpu.sync_copy(ci_hbm, mi)
        lane = lax.iota(jnp.int32, 16)
        res_v = jnp.full((16,), NEG_INF, jnp.float32)
        res_i = jnp.zeros((16,), jnp.int32)
        nv = ncand // 16
        for r in range(K):
            def body(c, carry):
                acc_v, acc_g, acc_c = carry
                v = mv[pl.ds(c * 16, 16)]
                gi = mi[pl.ds(c * 16, 16)]
                m = v > acc_v
                return (jnp.where(m, v, acc_v), jnp.where(m, gi, acc_g),
                        jnp.where(m, c, acc_c))

            acc_v, acc_g, acc_c = lax.fori_loop(
                0, nv, body,
                (jnp.full((16,), NEG_INF, jnp.float32),
                 jnp.zeros((16,), jnp.int32), jnp.zeros((16,), jnp.int32)))
            tmax = jnp.max(acc_v)
            eq = acc_v == tmax
            g = jnp.min(jnp.where(eq, acc_g, INT_MAX))
            pos = jnp.min(jnp.where(eq & (acc_g == g),
                                    acc_c * 16 + lane, INT_MAX))
            res_v = jnp.where(lane == r, tmax, res_v)
            res_i = jnp.where(lane == r, g, res_i)
            plsc.store_scatter(mv, [jnp.full((16,), pos, jnp.int32)],
                               jnp.full((16,), NEG_INF, jnp.float32),
                               mask=lane == 0)
        ov[...] = res_v
        oi[...] = res_i
        pltpu.sync_copy(ov.at[pl.ds(0, K)], ov_hbm)
        pltpu.sync_copy(oi.at[pl.ds(0, K)], oi_hbm)


def _merge(cv, ci):
    ncand = cv.shape[0]
    mesh = plsc.VectorSubcoreMesh(core_axis_name="c", subcore_axis_name="s")
    kern = pl.kernel(
        functools.partial(_merge_body, ncand),
        mesh=mesh,
        compiler_params=pltpu.CompilerParams(needs_layout_passes=False),
        out_type=(
            jax.ShapeDtypeStruct((K,), jnp.float32),
            jax.ShapeDtypeStruct((K,), jnp.int32),
        ),
        scratch_types=[
            pltpu.VMEM((ncand,), jnp.float32),
            pltpu.VMEM((ncand,), jnp.int32),
            pltpu.VMEM((16,), jnp.float32),
            pltpu.VMEM((16,), jnp.int32),
        ],
    )
    return kern(cv, ci)


def kernel(task_embedding, skill_matrix, top_k):
    te = jnp.reshape(task_embedding, (-1,))[:D]
    te_n = jnp.sqrt(jnp.sum(te * te))
    u = jax.lax.bitcast_convert_type(te, jnp.uint32)
    u = (u + 0x7FFF + ((u >> 16) & 1)) & jnp.uint32(0xFFFF0000)
    te_b = jax.lax.bitcast_convert_type(u, jnp.float32)
    te_s = te_b / jnp.maximum(te_n, EPS)
    sims = _sims_tc(skill_matrix, te_s)
    cv, ci = _tile_topk(sims)
    tv, ti = _merge(cv.reshape(-1), ci.reshape(-1))
    return tv, ti


# speedup vs baseline: 2.2626x; 1.7951x over previous
import functools

import jax
import jax.numpy as jnp
from jax import lax
from jax.experimental import pallas as pl
from jax.experimental.pallas import tpu as pltpu
from jax.experimental.pallas import tpu_sc as plsc

D = 64
K = 8
EPS = 1e-8
BLK = 8192
NEG_INF = float("-inf")
INT_MAX = 2**31 - 1


def _sims_body(nrows, te_ref, x_ref, o_ref):
    i = pl.program_id(0)
    x = x_ref[...]
    xt = x.T
    te = te_ref[...]
    d = lax.dot_general(te, xt, (((1,), (0,)), ((), ())),
                        precision=lax.Precision.DEFAULT)
    n2 = jnp.sum(xt * xt, axis=0)
    sims = d.reshape(x.shape[0]) / jnp.maximum(jnp.sqrt(n2), EPS)
    grow = i * x.shape[0] + lax.broadcasted_iota(jnp.int32, (x.shape[0],), 0)
    o_ref[...] = jnp.where(grow < nrows, sims, NEG_INF)


def _sims_tc(skill_matrix, te_row):
    n = skill_matrix.shape[0]
    grid = pl.cdiv(n, BLK)
    return pl.pallas_call(
        functools.partial(_sims_body, n),
        grid=(grid,),
        in_specs=[
            pl.BlockSpec((1, D), lambda i: (0, 0)),
            pl.BlockSpec((BLK, D), lambda i: (i, 0)),
        ],
        out_specs=pl.BlockSpec((BLK,), lambda i: (i,)),
        out_shape=jax.ShapeDtypeStruct((grid * BLK,), jnp.float32),
    )(te_row, skill_matrix)


def _tile_topk_body(nw, base, sims_hbm, cv_hbm, ci_hbm, chunk, ov, oi):
    info = plsc.get_sparse_core_info()
    nc = info.num_cores
    wid = lax.axis_index("s") * nc + lax.axis_index("c")
    off = wid * base
    pltpu.sync_copy(sims_hbm.at[pl.ds(off, base)], chunk)

    nv = base // 16
    lane = lax.iota(jnp.int32, 16)
    res_v = jnp.full((16,), NEG_INF, jnp.float32)
    res_i = jnp.zeros((16,), jnp.int32)

    for r in range(K):
        def body(c, carry):
            acc_v, acc_i = carry
            v = chunk[pl.ds(c * 16, 16)]
            m = v > acc_v
            return jnp.where(m, v, acc_v), jnp.where(m, c, acc_i)

        acc_v, acc_i = lax.fori_loop(
            0, nv,
            body,
            (jnp.full((16,), NEG_INF, jnp.float32), jnp.zeros((16,), jnp.int32)),
        )
        tmax = jnp.max(acc_v)
        gcand = jnp.where(acc_v == tmax, acc_i * 16 + lane, INT_MAX)
        g = jnp.min(gcand)
        res_v = jnp.where(lane == r, tmax, res_v)
        res_i = jnp.where(lane == r, g + off, res_i)
        plsc.store_scatter(chunk, [jnp.full((16,), g, jnp.int32)],
                           jnp.full((16,), NEG_INF, jnp.float32),
                           mask=lane == 0)

    ov[...] = res_v
    oi[...] = res_i
    pltpu.sync_copy(ov, cv_hbm.at[wid])
    pltpu.sync_copy(oi, ci_hbm.at[wid])


def _tile_topk(sims):
    n = sims.shape[0]
    info = plsc.get_sparse_core_info()
    nw = info.num_cores * info.num_subcores
    assert n % (nw * 16) == 0
    base = n // nw
    mesh = plsc.VectorSubcoreMesh(core_axis_name="c", subcore_axis_name="s")
    kern = pl.kernel(
        functools.partial(_tile_topk_body, nw, base),
        mesh=mesh,
        compiler_params=pltpu.CompilerParams(needs_layout_passes=False),
        out_type=(
            jax.ShapeDtypeStruct((nw, 16), jnp.float32),
            jax.ShapeDtypeStruct((nw, 16), jnp.int32),
        ),
        scratch_types=[
            pltpu.VMEM((base,), jnp.float32),
            pltpu.VMEM((16,), jnp.float32),
            pltpu.VMEM((16,), jnp.int32),
        ],
    )
    return kern(sims)


def _merge_body(ncand, cv_hbm, ci_hbm, ov_hbm, oi_hbm, mv, mi, ov, oi):
    info = plsc.get_sparse_core_info()
    nc = info.num_cores
    wid = lax.axis_index("s") * nc + lax.axis_index("c")

    @pl.when(wid == 0)
    def _():
        pltpu.sync_copy(cv_hbm, mv)
        pltpu.sync_copy(ci_hbm, mi)
        lane = lax.iota(jnp.int32, 16)
        res_v = jnp.full((16,), NEG_INF, jnp.float32)
        res_i = jnp.zeros((16,), jnp.int32)
        nv = ncand // 16
        for r in range(K):
            def body(c, carry):
                acc_v, acc_g, acc_c = carry
                v = mv[pl.ds(c * 16, 16)]
                gi = mi[pl.ds(c * 16, 16)]
                m = v > acc_v
                return (jnp.where(m, v, acc_v), jnp.where(m, gi, acc_g),
                        jnp.where(m, c, acc_c))

            acc_v, acc_g, acc_c = lax.fori_loop(
                0, nv, body,
                (jnp.full((16,), NEG_INF, jnp.float32),
                 jnp.zeros((16,), jnp.int32), jnp.zeros((16,), jnp.int32)))
            tmax = jnp.max(acc_v)
            eq = acc_v == tmax
            g = jnp.min(jnp.where(eq, acc_g, INT_MAX))
            pos = jnp.min(jnp.where(eq & (acc_g == g),
                                    acc_c * 16 + lane, INT_MAX))
            res_v = jnp.where(lane == r, tmax, res_v)
            res_i = jnp.where(lane == r, g, res_i)
            plsc.store_scatter(mv, [jnp.full((16,), pos, jnp.int32)],
                               jnp.full((16,), NEG_INF, jnp.float32),
                               mask=lane == 0)
        ov[...] = res_v
        oi[...] = res_i
        pltpu.sync_copy(ov.at[pl.ds(0, K)], ov_hbm)
        pltpu.sync_copy(oi.at[pl.ds(0, K)], oi_hbm)


def _merge(cv, ci):
    ncand = cv.shape[0]
    mesh = plsc.VectorSubcoreMesh(core_axis_name="c", subcore_axis_name="s")
    kern = pl.kernel(
        functools.partial(_merge_body, ncand),
        mesh=mesh,
        compiler_params=pltpu.CompilerParams(needs_layout_passes=False),
        out_type=(
            jax.ShapeDtypeStruct((K,), jnp.float32),
            jax.ShapeDtypeStruct((K,), jnp.int32),
        ),
        scratch_types=[
            pltpu.VMEM((ncand,), jnp.float32),
            pltpu.VMEM((ncand,), jnp.int32),
            pltpu.VMEM((16,), jnp.float32),
            pltpu.VMEM((16,), jnp.int32),
        ],
    )
    return kern(cv, ci)


def kernel(task_embedding, skill_matrix, top_k):
    te = jnp.reshape(task_embedding, (-1,))[:D]
    te_n = jnp.sqrt(jnp.sum(te * te))
    sims = _sims_tc(skill_matrix, te.reshape(1, D))
    cv, ci = _tile_topk(sims)
    tv, ti = _merge(cv.reshape(-1), ci.reshape(-1))
    return tv / jnp.maximum(te_n, EPS), ti


# speedup vs baseline: 2.6879x; 1.1880x over previous
import functools

import jax
import jax.numpy as jnp
from jax import lax
from jax.experimental import pallas as pl
from jax.experimental.pallas import tpu as pltpu
from jax.experimental.pallas import tpu_sc as plsc

D = 64
K = 8
EPS = 1e-8
BLK = 16384
SEG = 64
NEG_INF = float("-inf")
INT_MAX = 2**31 - 1


def _sims_body(nrows, te_ref, x_ref, o_ref):
    i = pl.program_id(0)
    x = x_ref[...]
    xt = x.T
    te = te_ref[...]
    d = lax.dot_general(te, xt, (((1,), (0,)), ((), ())),
                        precision=lax.Precision.DEFAULT)
    n2 = jnp.sum(xt * xt, axis=0)
    sims = d.reshape(x.shape[0]) / jnp.maximum(jnp.sqrt(n2), EPS)
    grow = i * x.shape[0] + lax.broadcasted_iota(jnp.int32, (x.shape[0],), 0)
    o_ref[...] = jnp.where(grow < nrows, sims, NEG_INF)


def _sims_tc(skill_matrix, te_row):
    n = skill_matrix.shape[0]
    grid = pl.cdiv(n, BLK)
    return pl.pallas_call(
        functools.partial(_sims_body, n),
        grid=(grid,),
        in_specs=[
            pl.BlockSpec((1, D), lambda i: (0, 0)),
            pl.BlockSpec((BLK, D), lambda i: (i, 0)),
        ],
        out_specs=pl.BlockSpec((BLK,), lambda i: (i,)),
        out_shape=jax.ShapeDtypeStruct((grid * BLK,), jnp.float32),
    )(te_row, skill_matrix)


def _tile_topk_body(nw, base, sims_hbm, cv_hbm, ci_hbm, chunk, segmax, segidx,
                    ov, oi):
    info = plsc.get_sparse_core_info()
    nc = info.num_cores
    wid = lax.axis_index("s") * nc + lax.axis_index("c")
    off = wid * base
    pltpu.sync_copy(sims_hbm.at[pl.ds(off, base)], chunk)

    ns = base // 16 // SEG
    lane = lax.iota(jnp.int32, 16)

    def scan_seg(s):
        def jbody(j4, c):
            acc_v, acc_i = c
            for u in range(4):
                cidx = s * SEG + j4 * 4 + u
                v = chunk[pl.ds(cidx * 16, 16)]
                m = v > acc_v
                acc_v = jnp.where(m, v, acc_v)
                acc_i = jnp.where(m, cidx, acc_i)
            return acc_v, acc_i

        acc_v, acc_i = lax.fori_loop(
            0, SEG // 4, jbody,
            (jnp.full((16,), NEG_INF, jnp.float32), jnp.zeros((16,), jnp.int32)))
        segmax[pl.ds(s * 16, 16)] = acc_v
        segidx[pl.ds(s * 16, 16)] = acc_i

    def pass0(s, carry):
        scan_seg(s)
        return carry

    lax.fori_loop(0, ns, pass0, 0)

    def round_body(r, carry):
        res_v, res_i = carry
        def seg_scan(s, c):
            acc_v, acc_s = c
            v = segmax[pl.ds(s * 16, 16)]
            m = v > acc_v
            return jnp.where(m, v, acc_v), jnp.where(m, s, acc_s)

        acc_v, acc_s = lax.fori_loop(
            0, ns, seg_scan,
            (jnp.full((16,), NEG_INF, jnp.float32), jnp.zeros((16,), jnp.int32)))
        tmax = jnp.max(acc_v)
        ctr = plsc.load_gather(segidx, [acc_s * 16 + lane])
        gcand = jnp.where(acc_v == tmax, ctr * 16 + lane, INT_MAX)
        g = jnp.min(gcand)
        res_v = jnp.where(lane == r, tmax, res_v)
        res_i = jnp.where(lane == r, g + off, res_i)
        plsc.store_scatter(chunk, [jnp.full((16,), g, jnp.int32)],
                           jnp.full((16,), NEG_INF, jnp.float32),
                           mask=lane == 0)
        scan_seg(lax.shift_right_logical(g, 10))
        return res_v, res_i

    res_v, res_i = lax.fori_loop(
        0, K, round_body,
        (jnp.full((16,), NEG_INF, jnp.float32), jnp.zeros((16,), jnp.int32)))

    ov[...] = res_v
    oi[...] = res_i
    pltpu.sync_copy(ov, cv_hbm.at[wid])
    pltpu.sync_copy(oi, ci_hbm.at[wid])


def _tile_topk(sims):
    n = sims.shape[0]
    info = plsc.get_sparse_core_info()
    nw = info.num_cores * info.num_subcores
    assert n % (nw * 16 * SEG) == 0
    base = n // nw
    mesh = plsc.VectorSubcoreMesh(core_axis_name="c", subcore_axis_name="s")
    kern = pl.kernel(
        functools.partial(_tile_topk_body, nw, base),
        mesh=mesh,
        compiler_params=pltpu.CompilerParams(needs_layout_passes=False),
        out_type=(
            jax.ShapeDtypeStruct((nw, 16), jnp.float32),
            jax.ShapeDtypeStruct((nw, 16), jnp.int32),
        ),
        scratch_types=[
            pltpu.VMEM((base,), jnp.float32),
            pltpu.VMEM((base // 16 // SEG * 16,), jnp.float32),
            pltpu.VMEM((base // 16 // SEG * 16,), jnp.int32),
            pltpu.VMEM((16,), jnp.float32),
            pltpu.VMEM((16,), jnp.int32),
        ],
    )
    return kern(sims)


def _merge_body(ncand, cv_hbm, ci_hbm, ov_hbm, oi_hbm, mv, mi, ov, oi):
    info = plsc.get_sparse_core_info()
    nc = info.num_cores
    wid = lax.axis_index("s") * nc + lax.axis_index("c")

    @pl.when(wid == 0)
    def _():
        pltpu.sync_copy(cv_hbm, mv)
        pltpu.sync_copy(ci_hbm, mi)
        lane = lax.iota(jnp.int32, 16)
        res_v = jnp.full((16,), NEG_INF, jnp.float32)
        res_i = jnp.zeros((16,), jnp.int32)
        nv = ncand // 16
        for r in range(K):
            def body(c, carry):
                acc_v, acc_g, acc_c = carry
                v = mv[pl.ds(c * 16, 16)]
                gi = mi[pl.ds(c * 16, 16)]
                m = v > acc_v
                return (jnp.where(m, v, acc_v), jnp.where(m, gi, acc_g),
                        jnp.where(m, c, acc_c))

            acc_v, acc_g, acc_c = lax.fori_loop(
                0, nv, body,
                (jnp.full((16,), NEG_INF, jnp.float32),
                 jnp.zeros((16,), jnp.int32), jnp.zeros((16,), jnp.int32)))
            tmax = jnp.max(acc_v)
            eq = acc_v == tmax
            g = jnp.min(jnp.where(eq, acc_g, INT_MAX))
            pos = jnp.min(jnp.where(eq & (acc_g == g),
                                    acc_c * 16 + lane, INT_MAX))
            res_v = jnp.where(lane == r, tmax, res_v)
            res_i = jnp.where(lane == r, g, res_i)
            plsc.store_scatter(mv, [jnp.full((16,), pos, jnp.int32)],
                               jnp.full((16,), NEG_INF, jnp.float32),
                               mask=lane == 0)
        ov[...] = res_v
        oi[...] = res_i
        pltpu.sync_copy(ov.at[pl.ds(0, K)], ov_hbm)
        pltpu.sync_copy(oi.at[pl.ds(0, K)], oi_hbm)


def _merge(cv, ci):
    ncand = cv.shape[0]
    mesh = plsc.VectorSubcoreMesh(core_axis_name="c", subcore_axis_name="s")
    kern = pl.kernel(
        functools.partial(_merge_body, ncand),
        mesh=mesh,
        compiler_params=pltpu.CompilerParams(needs_layout_passes=False),
        out_type=(
            jax.ShapeDtypeStruct((K,), jnp.float32),
            jax.ShapeDtypeStruct((K,), jnp.int32),
        ),
        scratch_types=[
            pltpu.VMEM((ncand,), jnp.float32),
            pltpu.VMEM((ncand,), jnp.int32),
            pltpu.VMEM((16,), jnp.float32),
            pltpu.VMEM((16,), jnp.int32),
        ],
    )
    return kern(cv, ci)


def kernel(task_embedding, skill_matrix, top_k):
    te = jnp.reshape(task_embedding, (-1,))[:D]
    te_n = jnp.sqrt(jnp.sum(te * te))
    sims = _sims_tc(skill_matrix, te.reshape(1, D))
    cv, ci = _tile_topk(sims)
    tv, ti = _merge(cv.reshape(-1), ci.reshape(-1))
    return tv / jnp.maximum(te_n, EPS), ti


# speedup vs baseline: 2.7713x; 1.0310x over previous
import functools

import jax
import jax.numpy as jnp
from jax import lax
from jax.experimental import pallas as pl
from jax.experimental.pallas import tpu as pltpu
from jax.experimental.pallas import tpu_sc as plsc

D = 64
K = 8
EPS = 1e-8
BLK = 16384
SEG = 64
NEG_INF = float("-inf")
INT_MAX = 2**31 - 1


def _sims_half(i, half, nrows, te, x):
    xt = x.T
    d = lax.dot_general(te, xt, (((1,), (0,)), ((), ())),
                        precision=lax.Precision.DEFAULT)
    n2 = jnp.sum(xt * xt, axis=0)
    sims = d.reshape(x.shape[0]) / jnp.maximum(jnp.sqrt(n2), EPS)
    grow = ((2 * i + half) * x.shape[0]
            + lax.broadcasted_iota(jnp.int32, (x.shape[0],), 0))
    return jnp.where(grow < nrows, sims, NEG_INF)


def _sims_body(nrows, te_ref, xa_ref, xb_ref, o_ref):
    i = pl.program_id(0)
    te = te_ref[...]
    o_ref[pl.ds(0, BLK)] = _sims_half(i, 0, nrows, te, xa_ref[...])
    o_ref[pl.ds(BLK, BLK)] = _sims_half(i, 1, nrows, te, xb_ref[...])


def _sims_tc(skill_matrix, te_row):
    n = skill_matrix.shape[0]
    grid = pl.cdiv(n, 2 * BLK)
    return pl.pallas_call(
        functools.partial(_sims_body, n),
        grid=(grid,),
        in_specs=[
            pl.BlockSpec((1, D), lambda i: (0, 0)),
            pl.BlockSpec((BLK, D), lambda i: (2 * i, 0)),
            pl.BlockSpec((BLK, D), lambda i: (2 * i + 1, 0)),
        ],
        out_specs=pl.BlockSpec((2 * BLK,), lambda i: (i,)),
        out_shape=jax.ShapeDtypeStruct((grid * 2 * BLK,), jnp.float32),
    )(te_row, skill_matrix, skill_matrix)


def _tile_topk_body(nw, base, sims_hbm, cv_hbm, ci_hbm, chunk, segmax, segidx,
                    ov, oi):
    info = plsc.get_sparse_core_info()
    nc = info.num_cores
    wid = lax.axis_index("s") * nc + lax.axis_index("c")
    off = wid * base
    pltpu.sync_copy(sims_hbm.at[pl.ds(off, base)], chunk)

    ns = base // 16 // SEG
    lane = lax.iota(jnp.int32, 16)

    def scan_seg(s):
        def jbody(j4, c):
            acc_v, acc_i = c
            for u in range(4):
                cidx = s * SEG + j4 * 4 + u
                v = chunk[pl.ds(cidx * 16, 16)]
                m = v > acc_v
                acc_v = jnp.where(m, v, acc_v)
                acc_i = jnp.where(m, cidx, acc_i)
            return acc_v, acc_i

        acc_v, acc_i = lax.fori_loop(
            0, SEG // 4, jbody,
            (jnp.full((16,), NEG_INF, jnp.float32), jnp.zeros((16,), jnp.int32)))
        segmax[pl.ds(s * 16, 16)] = acc_v
        segidx[pl.ds(s * 16, 16)] = acc_i

    def pass0(s, carry):
        scan_seg(s)
        return carry

    lax.fori_loop(0, ns, pass0, 0)

    def round_body(r, carry):
        res_v, res_i = carry
        def seg_scan(s, c):
            acc_v, acc_s = c
            v = segmax[pl.ds(s * 16, 16)]
            m = v > acc_v
            return jnp.where(m, v, acc_v), jnp.where(m, s, acc_s)

        acc_v, acc_s = lax.fori_loop(
            0, ns, seg_scan,
            (jnp.full((16,), NEG_INF, jnp.float32), jnp.zeros((16,), jnp.int32)))
        tmax = jnp.max(acc_v)
        ctr = plsc.load_gather(segidx, [acc_s * 16 + lane])
        gcand = jnp.where(acc_v == tmax, ctr * 16 + lane, INT_MAX)
        g = jnp.min(gcand)
        res_v = jnp.where(lane == r, tmax, res_v)
        res_i = jnp.where(lane == r, g + off, res_i)
        plsc.store_scatter(chunk, [jnp.full((16,), g, jnp.int32)],
                           jnp.full((16,), NEG_INF, jnp.float32),
                           mask=lane == 0)
        scan_seg(lax.shift_right_logical(g, 10))
        return res_v, res_i

    res_v, res_i = lax.fori_loop(
        0, K, round_body,
        (jnp.full((16,), NEG_INF, jnp.float32), jnp.zeros((16,), jnp.int32)))

    ov[...] = res_v
    oi[...] = res_i
    pltpu.sync_copy(ov, cv_hbm.at[wid])
    pltpu.sync_copy(oi, ci_hbm.at[wid])


def _tile_topk(sims):
    n = sims.shape[0]
    info = plsc.get_sparse_core_info()
    nw = info.num_cores * info.num_subcores
    assert n % (nw * 16 * SEG) == 0
    base = n // nw
    mesh = plsc.VectorSubcoreMesh(core_axis_name="c", subcore_axis_name="s")
    kern = pl.kernel(
        functools.partial(_tile_topk_body, nw, base),
        mesh=mesh,
        compiler_params=pltpu.CompilerParams(needs_layout_passes=False),
        out_type=(
            jax.ShapeDtypeStruct((nw, 16), jnp.float32),
            jax.ShapeDtypeStruct((nw, 16), jnp.int32),
        ),
        scratch_types=[
            pltpu.VMEM((base,), jnp.float32),
            pltpu.VMEM((base // 16 // SEG * 16,), jnp.float32),
            pltpu.VMEM((base // 16 // SEG * 16,), jnp.int32),
            pltpu.VMEM((16,), jnp.float32),
            pltpu.VMEM((16,), jnp.int32),
        ],
    )
    return kern(sims)


def _merge_body(ncand, cv_hbm, ci_hbm, ov_hbm, oi_hbm, mv, mi, ov, oi):
    info = plsc.get_sparse_core_info()
    nc = info.num_cores
    wid = lax.axis_index("s") * nc + lax.axis_index("c")

    @pl.when(wid == 0)
    def _():
        pltpu.sync_copy(cv_hbm, mv)
        pltpu.sync_copy(ci_hbm, mi)
        lane = lax.iota(jnp.int32, 16)
        res_v = jnp.full((16,), NEG_INF, jnp.float32)
        res_i = jnp.zeros((16,), jnp.int32)
        nv = ncand // 16
        for r in range(K):
            def body(c, carry):
                acc_v, acc_g, acc_c = carry
                v = mv[pl.ds(c * 16, 16)]
                gi = mi[pl.ds(c * 16, 16)]
                m = v > acc_v
                return (jnp.where(m, v, acc_v), jnp.where(m, gi, acc_g),
                        jnp.where(m, c, acc_c))

            acc_v, acc_g, acc_c = lax.fori_loop(
                0, nv, body,
                (jnp.full((16,), NEG_INF, jnp.float32),
                 jnp.zeros((16,), jnp.int32), jnp.zeros((16,), jnp.int32)))
            tmax = jnp.max(acc_v)
            eq = acc_v == tmax
            g = jnp.min(jnp.where(eq, acc_g, INT_MAX))
            pos = jnp.min(jnp.where(eq & (acc_g == g),
                                    acc_c * 16 + lane, INT_MAX))
            res_v = jnp.where(lane == r, tmax, res_v)
            res_i = jnp.where(lane == r, g, res_i)
            plsc.store_scatter(mv, [jnp.full((16,), pos, jnp.int32)],
                               jnp.full((16,), NEG_INF, jnp.float32),
                               mask=lane == 0)
        ov[...] = res_v
        oi[...] = res_i
        pltpu.sync_copy(ov.at[pl.ds(0, K)], ov_hbm)
        pltpu.sync_copy(oi.at[pl.ds(0, K)], oi_hbm)


def _merge(cv, ci):
    ncand = cv.shape[0]
    mesh = plsc.VectorSubcoreMesh(core_axis_name="c", subcore_axis_name="s")
    kern = pl.kernel(
        functools.partial(_merge_body, ncand),
        mesh=mesh,
        compiler_params=pltpu.CompilerParams(needs_layout_passes=False),
        out_type=(
            jax.ShapeDtypeStruct((K,), jnp.float32),
            jax.ShapeDtypeStruct((K,), jnp.int32),
        ),
        scratch_types=[
            pltpu.VMEM((ncand,), jnp.float32),
            pltpu.VMEM((ncand,), jnp.int32),
            pltpu.VMEM((16,), jnp.float32),
            pltpu.VMEM((16,), jnp.int32),
        ],
    )
    return kern(cv, ci)


def kernel(task_embedding, skill_matrix, top_k):
    te = jnp.reshape(task_embedding, (-1,))[:D]
    te_n = jnp.sqrt(jnp.sum(te * te))
    sims = _sims_tc(skill_matrix, te.reshape(1, D))
    cv, ci = _tile_topk(sims)
    tv, ti = _merge(cv.reshape(-1), ci.reshape(-1))
    return tv / jnp.maximum(te_n, EPS), ti
